# baseline (speedup 1.0000x reference)
import functools

import jax
import jax.numpy as jnp
from jax import lax
from jax.experimental import pallas as pl
from jax.experimental.pallas import tpu as pltpu
from jax.experimental.pallas import tpu_sc as plsc


_CH = 64
_NS = 3


def _sc_gather(xt, ge0, ge1, ge2, ge3):
    E = ge0.shape[0]
    F = xt.shape[1]
    info = plsc.get_sparse_core_info()
    NW = info.num_cores * info.num_subcores
    per_w = E // NW
    assert per_w * NW == E and per_w % 8 == 0
    nfull = per_w // _CH
    tail = per_w - nfull * _CH
    assert tail % 8 == 0 and 0 < tail <= 128
    mesh = plsc.VectorSubcoreMesh(core_axis_name="c", subcore_axis_name="s")

    @functools.partial(
        pl.kernel,
        mesh=mesh,
        out_type=jax.ShapeDtypeStruct((4, E, F), jnp.float32),
        scratch_types=[
            pltpu.VMEM((per_w,), jnp.int32),
            pltpu.VMEM((per_w,), jnp.int32),
            pltpu.VMEM((per_w,), jnp.int32),
            pltpu.VMEM((per_w,), jnp.int32),
            pltpu.VMEM((_NS, 4, _CH, F), jnp.float32),
            pltpu.VMEM((4, tail, F), jnp.float32),
            pltpu.SemaphoreType.DMA((_NS,)),
            pltpu.SemaphoreType.DMA((_NS,)),
        ],
    )
    def k(xt_hbm, ge0_h, ge1_h, ge2_h, ge3_h, out_hbm, i0_v, i1_v, i2_v,
          i3_v, rows_v, trows_v, gsem, wsem):
        wid = lax.axis_index("s") * info.num_cores + lax.axis_index("c")
        w_base = pl.multiple_of(wid * per_w, 8)
        ge_h = (ge0_h, ge1_h, ge2_h, ge3_h)
        idx_v = (i0_v, i1_v, i2_v, i3_v)

        for t in range(4):
            pltpu.sync_copy(ge_h[t].at[pl.ds(w_base, per_w)], idx_v[t])

        def fire_gather(i, s):
            for t in range(4):
                pltpu.async_copy(
                    xt_hbm.at[idx_v[t].at[pl.ds(i * _CH, _CH)]],
                    rows_v.at[s, t],
                    gsem.at[s],
                )

        def wait_gather(i, s):
            for t in range(4):
                pltpu.make_async_copy(
                    xt_hbm.at[idx_v[t].at[pl.ds(i * _CH, _CH)]],
                    rows_v.at[s, t],
                    gsem.at[s],
                ).wait()

        def fire_wb(i, s):
            base = pl.multiple_of(w_base + i * _CH, 8)
            for t in range(4):
                pltpu.async_copy(
                    rows_v.at[s, t], out_hbm.at[t, pl.ds(base, _CH)], wsem.at[s]
                )

        def wait_wb(i, s):
            base = pl.multiple_of(w_base + i * _CH, 8)
            for t in range(4):
                pltpu.make_async_copy(
                    rows_v.at[s, t], out_hbm.at[t, pl.ds(base, _CH)], wsem.at[s]
                ).wait()

        def body(i, carry):
            s = lax.rem(i, _NS)
            sp = lax.rem(i + _NS - 1, _NS)

            @pl.when(i >= _NS)
            def _():
                wait_wb(i - _NS, s)

            @pl.when(i < nfull)
            def _():
                fire_gather(i, s)

            @pl.when(i >= 1)
            def _():
                wait_gather(i - 1, sp)
                fire_wb(i - 1, sp)

            return carry

        lax.fori_loop(0, nfull + 1, body, 0)
        for j in range(nfull - _NS + 1, nfull):
            wait_wb(j, j % _NS)

        tbase = pl.multiple_of(w_base + nfull * _CH, 8)
        tcps = [
            pltpu.async_copy(
                xt_hbm.at[idx_v[t].at[pl.ds(nfull * _CH, tail)]],
                trows_v.at[t],
                gsem.at[0],
            )
            for t in range(4)
        ]
        for cp in tcps:
            cp.wait()
        for t in range(4):
            pltpu.sync_copy(trows_v.at[t], out_hbm.at[t, pl.ds(tbase, tail)])

    return k(xt, ge0, ge1, ge2, ge3)


def _tc_combine(xt, g, W5, b2):
    E, F = xt.shape
    O = W5.shape[2]
    BE = 8000
    grid = (E // BE,)

    def body(xt_ref, g_ref, w_ref, b_ref, out_ref):
        x0 = xt_ref[...]
        g1, g2, g3, g4 = g_ref[0], g_ref[1], g_ref[2], g_ref[3]
        s1 = g1 + g3
        s2 = g2 + g4
        a1 = jnp.abs(g1 - g3)
        a2 = jnp.abs(g2 - g4)
        w = w_ref[...]
        acc = jnp.dot(x0, w[0], preferred_element_type=jnp.float32)
        acc += jnp.dot(s1, w[1], preferred_element_type=jnp.float32)
        acc += jnp.dot(s2, w[2], preferred_element_type=jnp.float32)
        acc += jnp.dot(a1, w[3], preferred_element_type=jnp.float32)
        acc += jnp.dot(a2, w[4], preferred_element_type=jnp.float32)
        out_ref[...] = acc + b_ref[...]

    return pl.pallas_call(
        body,
        grid=grid,
        in_specs=[
            pl.BlockSpec((BE, F), lambda i: (i, 0)),
            pl.BlockSpec((4, BE, F), lambda i: (0, i, 0)),
            pl.BlockSpec((5, F, O), lambda i: (0, 0, 0)),
            pl.BlockSpec((1, O), lambda i: (0, 0)),
        ],
        out_specs=pl.BlockSpec((BE, O), lambda i: (i, 0)),
        out_shape=jax.ShapeDtypeStruct((E, O), jnp.float32),
    )(xt, g, W5, b2)


def kernel(x, gemm_edges, W, b):
    xt = x[0].T
    W5 = jnp.transpose(W[:, :, 0, :], (2, 1, 0))
    g = _sc_gather(xt, gemm_edges[:, 0], gemm_edges[:, 1],
                   gemm_edges[:, 2], gemm_edges[:, 3])
    out = _tc_combine(xt, g, W5, b[None, :])
    return jnp.transpose(out)[None, :, :, None]


# speedup vs baseline: 1.0667x; 1.0667x over previous
import functools

import jax
import jax.numpy as jnp
from jax import lax
from jax.experimental import pallas as pl
from jax.experimental.pallas import tpu as pltpu
from jax.experimental.pallas import tpu_sc as plsc


_CH = 64
_NS = 3


def _sc_gather(xt, ge0, ge1, ge2, ge3):
    E = ge0.shape[0]
    F = xt.shape[1]
    info = plsc.get_sparse_core_info()
    NW = info.num_cores * info.num_subcores
    per_w = E // NW
    assert per_w * NW == E and per_w % 8 == 0
    nfull = per_w // _CH
    tail = per_w - nfull * _CH
    assert tail % 8 == 0 and 0 < tail <= 128
    mesh = plsc.VectorSubcoreMesh(core_axis_name="c", subcore_axis_name="s")

    @functools.partial(
        pl.kernel,
        mesh=mesh,
        out_type=jax.ShapeDtypeStruct((4, E, F), jnp.float32),
        scratch_types=[
            pltpu.VMEM((per_w,), jnp.int32),
            pltpu.VMEM((per_w,), jnp.int32),
            pltpu.VMEM((per_w,), jnp.int32),
            pltpu.VMEM((per_w,), jnp.int32),
            pltpu.VMEM((_NS, 4, _CH, F), jnp.float32),
            pltpu.VMEM((4, tail, F), jnp.float32),
            pltpu.SemaphoreType.DMA((_NS,)),
            pltpu.SemaphoreType.DMA((_NS,)),
        ],
    )
    def k(xt_hbm, ge0_h, ge1_h, ge2_h, ge3_h, out_hbm, i0_v, i1_v, i2_v,
          i3_v, rows_v, trows_v, gsem, wsem):
        wid = lax.axis_index("s") * info.num_cores + lax.axis_index("c")
        w_base = pl.multiple_of(wid * per_w, 8)
        ge_h = (ge0_h, ge1_h, ge2_h, ge3_h)
        idx_v = (i0_v, i1_v, i2_v, i3_v)

        for t in range(4):
            pltpu.sync_copy(ge_h[t].at[pl.ds(w_base, per_w)], idx_v[t])

        def fire_gather(i, s):
            for t in range(4):
                pltpu.async_copy(
                    xt_hbm.at[idx_v[t].at[pl.ds(i * _CH, _CH)]],
                    rows_v.at[s, t],
                    gsem.at[s],
                )

        def wait_gather(i, s):
            for t in range(4):
                pltpu.make_async_copy(
                    xt_hbm.at[idx_v[t].at[pl.ds(i * _CH, _CH)]],
                    rows_v.at[s, t],
                    gsem.at[s],
                ).wait()

        def fire_wb(i, s):
            base = pl.multiple_of(w_base + i * _CH, 8)
            for t in range(4):
                pltpu.async_copy(
                    rows_v.at[s, t], out_hbm.at[t, pl.ds(base, _CH)], wsem.at[s]
                )

        def wait_wb(i, s):
            base = pl.multiple_of(w_base + i * _CH, 8)
            for t in range(4):
                pltpu.make_async_copy(
                    rows_v.at[s, t], out_hbm.at[t, pl.ds(base, _CH)], wsem.at[s]
                ).wait()

        def body(i, carry):
            s = lax.rem(i, _NS)
            sp = lax.rem(i + _NS - 1, _NS)

            @pl.when(i >= _NS)
            def _():
                wait_wb(i - _NS, s)

            @pl.when(i < nfull)
            def _():
                fire_gather(i, s)

            @pl.when(i >= 1)
            def _():
                wait_gather(i - 1, sp)
                fire_wb(i - 1, sp)

            return carry

        lax.fori_loop(0, nfull + 1, body, 0)
        for j in range(nfull - _NS + 1, nfull):
            wait_wb(j, j % _NS)

        tbase = pl.multiple_of(w_base + nfull * _CH, 8)
        tcps = [
            pltpu.async_copy(
                xt_hbm.at[idx_v[t].at[pl.ds(nfull * _CH, tail)]],
                trows_v.at[t],
                gsem.at[0],
            )
            for t in range(4)
        ]
        for cp in tcps:
            cp.wait()
        for t in range(4):
            pltpu.sync_copy(trows_v.at[t], out_hbm.at[t, pl.ds(tbase, tail)])

    return k(xt, ge0, ge1, ge2, ge3)


def _tc_combine(xt, g, W5, b2):
    E, F = xt.shape
    O = W5.shape[2]
    BE = 6400
    grid = (E // BE,)

    def body(xt_ref, g_ref, w_ref, b_ref, out_ref):
        x0 = xt_ref[...]
        g1, g2, g3, g4 = g_ref[0], g_ref[1], g_ref[2], g_ref[3]
        s1 = g1 + g3
        s2 = g2 + g4
        a1 = jnp.abs(g1 - g3)
        a2 = jnp.abs(g2 - g4)
        w = w_ref[...]
        acc = jnp.dot(x0, w[0], preferred_element_type=jnp.float32)
        acc += jnp.dot(s1, w[1], preferred_element_type=jnp.float32)
        acc += jnp.dot(s2, w[2], preferred_element_type=jnp.float32)
        acc += jnp.dot(a1, w[3], preferred_element_type=jnp.float32)
        acc += jnp.dot(a2, w[4], preferred_element_type=jnp.float32)
        out_ref[...] = jnp.transpose(acc) + b_ref[...]

    return pl.pallas_call(
        body,
        grid=grid,
        in_specs=[
            pl.BlockSpec((BE, F), lambda i: (i, 0)),
            pl.BlockSpec((4, BE, F), lambda i: (0, i, 0)),
            pl.BlockSpec((5, F, O), lambda i: (0, 0, 0)),
            pl.BlockSpec((O, 1), lambda i: (0, 0)),
        ],
        out_specs=pl.BlockSpec((O, BE), lambda i: (0, i)),
        out_shape=jax.ShapeDtypeStruct((O, E), jnp.float32),
    )(xt, g, W5, b2)


def kernel(x, gemm_edges, W, b):
    xt = x[0].T
    W5 = jnp.transpose(W[:, :, 0, :], (2, 1, 0))
    g = _sc_gather(xt, gemm_edges[:, 0], gemm_edges[:, 1],
                   gemm_edges[:, 2], gemm_edges[:, 3])
    out = _tc_combine(xt, g, W5, b[:, None])
    return out[None, :, :, None]
